# Initial kernel scaffold; baseline (speedup 1.0000x reference)
import jax
import jax.numpy as jnp
from jax.experimental import pallas as pl


def kernel(avg_features, W, b, embed_table):
    raise NotImplementedError("write your pallas kernel here")


# speedup vs baseline: 1.7849x; 1.7849x over previous
import functools

import jax
import jax.numpy as jnp
from jax import lax
from jax.experimental import pallas as pl
from jax.experimental.pallas import tpu as pltpu
from jax.experimental.pallas import tpu_sc as plsc

NUM_CLASSES = 210
SEM_DIM = 512
FC_IN = 2048
BATCH = 16384
K = 10


BM = 1024


def _tc_body(avg_ref, w_ref, b_ref, tags_ref, idx_ref):
    avg = avg_ref[...]
    w = w_ref[...]
    tags = lax.dot_general(
        avg, w,
        dimension_numbers=(((1,), (1,)), ((), ())),
        preferred_element_type=jnp.float32,
    ) + b_ref[...]
    tags_ref[...] = tags

    iota = lax.broadcasted_iota(jnp.int32, (BM, NUM_CLASSES), 1)
    work = tags
    cols = []
    for _ in range(K):
        m = jnp.max(work, axis=1, keepdims=True)
        cand = jnp.where(work == m, iota, NUM_CLASSES)
        a = jnp.min(cand, axis=1, keepdims=True)
        cols.append(a)
        work = jnp.where(iota == a, -jnp.inf, work)
    idx_ref[...] = jnp.concatenate(cols, axis=1)


def _tc_call(avg_features, W, b):
    grid = BATCH // BM
    return pl.pallas_call(
        _tc_body,
        grid=(grid,),
        in_specs=[
            pl.BlockSpec((BM, FC_IN), lambda i: (i, 0)),
            pl.BlockSpec((NUM_CLASSES, FC_IN), lambda i: (0, 0)),
            pl.BlockSpec((1, NUM_CLASSES), lambda i: (0, 0)),
        ],
        out_specs=[
            pl.BlockSpec((BM, NUM_CLASSES), lambda i: (i, 0)),
            pl.BlockSpec((BM, K), lambda i: (i, 0)),
        ],
        out_shape=[
            jax.ShapeDtypeStruct((BATCH, NUM_CLASSES), jnp.float32),
            jax.ShapeDtypeStruct((BATCH, K), jnp.int32),
        ],
    )(avg_features, W, b.reshape(1, NUM_CLASSES))


_SC_CHUNK = 64


def _sc_gather_kernel(table_hbm, idx_hbm, out_hbm, idx_v, rows_v, sem,
                      *, rows_per_worker, num_cores):
    wid = lax.axis_index("s") * num_cores + lax.axis_index("c")
    base = wid * rows_per_worker
    nchunks = rows_per_worker // _SC_CHUNK

    def body(i, carry):
        off = base + i * _SC_CHUNK
        pltpu.sync_copy(idx_hbm.at[pl.ds(off, _SC_CHUNK)], idx_v)
        pltpu.async_copy(table_hbm.at[idx_v], rows_v, sem).wait()
        pltpu.sync_copy(rows_v, out_hbm.at[pl.ds(off, _SC_CHUNK)])
        return carry

    lax.fori_loop(0, nchunks, body, 0)


def _sc_gather(embed_table, idx_flat):
    info = plsc.get_sparse_core_info()
    nw = info.num_cores * info.num_subcores
    total = BATCH * K
    rows_per_worker = total // nw
    mesh = plsc.VectorSubcoreMesh(core_axis_name="c", subcore_axis_name="s")
    kern = pl.kernel(
        functools.partial(_sc_gather_kernel,
                          rows_per_worker=rows_per_worker,
                          num_cores=info.num_cores),
        out_type=jax.ShapeDtypeStruct((total, SEM_DIM), jnp.float32),
        mesh=mesh,
        scratch_types=[
            pltpu.VMEM((_SC_CHUNK,), jnp.int32),
            pltpu.VMEM((_SC_CHUNK, SEM_DIM), jnp.float32),
            pltpu.SemaphoreType.DMA,
        ],
    )
    return kern(embed_table, idx_flat)


def kernel(avg_features, W, b, embed_table):
    tags, idx = _tc_call(avg_features, W, b)
    idx_flat = idx.reshape(-1)
    semantic = _sc_gather(embed_table, idx_flat)
    return tags, semantic.reshape(BATCH, K, SEM_DIM)


# speedup vs baseline: 2.6439x; 1.4812x over previous
import functools

import jax
import jax.numpy as jnp
from jax import lax
from jax.experimental import pallas as pl
from jax.experimental.pallas import tpu as pltpu
from jax.experimental.pallas import tpu_sc as plsc

NUM_CLASSES = 210
SEM_DIM = 512
FC_IN = 2048
BATCH = 16384
K = 10


BM = 1024


def _tc_body(avg_ref, w_ref, b_ref, tags_ref, idx_ref):
    avg = avg_ref[...]
    w = w_ref[...]
    tags = lax.dot_general(
        avg, w,
        dimension_numbers=(((1,), (1,)), ((), ())),
        preferred_element_type=jnp.float32,
    ) + b_ref[...]
    tags_ref[...] = tags

    iota = lax.broadcasted_iota(jnp.int32, (BM, NUM_CLASSES), 1)
    work = tags
    cols = []
    for _ in range(K):
        m = jnp.max(work, axis=1, keepdims=True)
        cand = jnp.where(work == m, iota, NUM_CLASSES)
        a = jnp.min(cand, axis=1, keepdims=True)
        cols.append(a)
        work = jnp.where(iota == a, -jnp.inf, work)
    idx_ref[...] = jnp.concatenate(cols, axis=1)


def _tc_call(avg_features, W, b):
    grid = BATCH // BM
    return pl.pallas_call(
        _tc_body,
        grid=(grid,),
        in_specs=[
            pl.BlockSpec((BM, FC_IN), lambda i: (i, 0)),
            pl.BlockSpec((NUM_CLASSES, FC_IN), lambda i: (0, 0)),
            pl.BlockSpec((1, NUM_CLASSES), lambda i: (0, 0)),
        ],
        out_specs=[
            pl.BlockSpec((BM, NUM_CLASSES), lambda i: (i, 0)),
            pl.BlockSpec((BM, K), lambda i: (i, 0)),
        ],
        out_shape=[
            jax.ShapeDtypeStruct((BATCH, NUM_CLASSES), jnp.float32),
            jax.ShapeDtypeStruct((BATCH, K), jnp.int32),
        ],
    )(avg_features, W, b.reshape(1, NUM_CLASSES))


_NB = 8
_NIDX = _NB * K
_NCH = _NIDX * 4


def _sc_gather_kernel(table_hbm, idx_hbm, out_hbm, idx_v, idx4_v, rv, sem,
                      *, rows_per_worker, num_cores):
    wid = lax.axis_index("s") * num_cores + lax.axis_index("c")
    b_base = wid * rows_per_worker
    nchunks = rows_per_worker // _NB
    lane = lax.iota(jnp.int32, 16)

    def body(i, carry):
        b0 = b_base + i * _NB
        pltpu.sync_copy(idx_hbm.at[pl.ds(b0 * K, _NIDX)], idx_v)
        for g in range(_NCH // 16):
            j = lax.shift_right_logical(lane, 2) + (g * 4)
            src = plsc.load_gather(idx_v, [j])
            idx4_v[pl.ds(g * 16, 16)] = (
                lax.shift_left(src, 2) + lax.bitwise_and(lane, 3))
        cps = []
        for p in range(0, _NCH, 128):
            n = min(128, _NCH - p)
            cps.append(pltpu.async_copy(
                table_hbm.at[idx4_v.at[pl.ds(p, n)]],
                rv.at[pl.ds(p, n)], sem))
        for cp in cps:
            cp.wait()
        pltpu.sync_copy(rv.reshape(_NB, K, SEM_DIM),
                        out_hbm.at[pl.ds(b0, _NB), :, :])
        return carry

    lax.fori_loop(0, nchunks, body, 0)


def _sc_gather(table840, idx_flat):
    info = plsc.get_sparse_core_info()
    nw = info.num_cores * info.num_subcores
    rows_per_worker = BATCH // nw
    mesh = plsc.VectorSubcoreMesh(core_axis_name="c", subcore_axis_name="s")
    kern = pl.kernel(
        functools.partial(_sc_gather_kernel,
                          rows_per_worker=rows_per_worker,
                          num_cores=info.num_cores),
        out_type=jax.ShapeDtypeStruct((BATCH, K, SEM_DIM), jnp.float32),
        mesh=mesh,
        scratch_types=[
            pltpu.VMEM((_NIDX,), jnp.int32),
            pltpu.VMEM((_NCH,), jnp.int32),
            pltpu.VMEM((_NCH, 128), jnp.float32),
            pltpu.SemaphoreType.DMA,
        ],
        compiler_params=pltpu.CompilerParams(use_tc_tiling_on_sc=True,
                                             needs_layout_passes=False),
    )
    return kern(table840, idx_flat)


def kernel(avg_features, W, b, embed_table):
    tags, idx = _tc_call(avg_features, W, b)
    idx_flat = idx.reshape(-1)
    table840 = embed_table.reshape(NUM_CLASSES * 4, SEM_DIM // 4)
    semantic = _sc_gather(table840, idx_flat)
    return tags, semantic


# speedup vs baseline: 4.4080x; 1.6673x over previous
import functools

import jax
import jax.numpy as jnp
from jax import lax
from jax.experimental import pallas as pl
from jax.experimental.pallas import tpu as pltpu
from jax.experimental.pallas import tpu_sc as plsc

NUM_CLASSES = 210
SEM_DIM = 512
FC_IN = 2048
BATCH = 16384
K = 10


BM = 1024


def _tc_body(avg_ref, w_ref, b_ref, tags_ref, idx_ref):
    avg = avg_ref[...]
    w = w_ref[...]
    tags = lax.dot_general(
        w, avg,
        dimension_numbers=(((1,), (1,)), ((), ())),
        preferred_element_type=jnp.float32,
    ) + b_ref[...]
    tags_ref[...] = tags

    iota = lax.broadcasted_iota(jnp.int32, (NUM_CLASSES, BM), 0)
    work = tags
    rows = []
    for _ in range(K):
        m = jnp.max(work, axis=0, keepdims=True)
        cand = jnp.where(work == m, iota, NUM_CLASSES)
        a = jnp.min(cand, axis=0, keepdims=True)
        rows.append(a)
        work = jnp.where(iota == a, -jnp.inf, work)
    idx_ref[...] = jnp.concatenate(rows, axis=0)


def _tc_call(avg_features, W, b):
    grid = BATCH // BM
    return pl.pallas_call(
        _tc_body,
        grid=(grid,),
        in_specs=[
            pl.BlockSpec((BM, FC_IN), lambda i: (i, 0)),
            pl.BlockSpec((NUM_CLASSES, FC_IN), lambda i: (0, 0)),
            pl.BlockSpec((NUM_CLASSES, 1), lambda i: (0, 0)),
        ],
        out_specs=[
            pl.BlockSpec((NUM_CLASSES, BM), lambda i: (0, i)),
            pl.BlockSpec((K, BM), lambda i: (0, i)),
        ],
        out_shape=[
            jax.ShapeDtypeStruct((NUM_CLASSES, BATCH), jnp.float32),
            jax.ShapeDtypeStruct((K, BATCH), jnp.int32),
        ],
    )(avg_features, W, b.reshape(NUM_CLASSES, 1))


_NBR = 128
_NCH = _NBR * 4


def _sc_gather_kernel(table_hbm, idx_hbm, out_hbm, idx_v, idx4_v, rv, sem,
                      *, rows_per_worker, num_cores):
    wid = lax.axis_index("s") * num_cores + lax.axis_index("c")
    b_base = wid * rows_per_worker
    nchunks = rows_per_worker // _NBR
    lane = lax.iota(jnp.int32, 16)

    def body(i, carry):
        b0 = b_base + i * _NBR
        pltpu.sync_copy(idx_hbm.at[:, pl.ds(b0, _NBR)], idx_v)

        def kbody(k, kcarry):
            for g in range(_NCH // 16):
                bl = lax.shift_right_logical(lane, 2) + (g * 4)
                kv = lax.bitwise_and(lane, 0) + k
                src = plsc.load_gather(idx_v, [kv, bl])
                idx4_v[pl.ds(g * 16, 16)] = (
                    lax.shift_left(src, 2) + lax.bitwise_and(lane, 3))
            cps = []
            for p in range(0, _NCH, 128):
                cps.append(pltpu.async_copy(
                    table_hbm.at[idx4_v.at[pl.ds(p, 128)]],
                    rv.at[pl.ds(p, 128)], sem))
            for cp in cps:
                cp.wait()
            pltpu.sync_copy(rv.reshape(_NBR, SEM_DIM),
                            out_hbm.at[k, pl.ds(b0, _NBR), :])
            return kcarry

        lax.fori_loop(0, K, kbody, 0)
        return carry

    lax.fori_loop(0, nchunks, body, 0)


def _sc_gather(table840, idx_t):
    info = plsc.get_sparse_core_info()
    nw = info.num_cores * info.num_subcores
    rows_per_worker = BATCH // nw
    mesh = plsc.VectorSubcoreMesh(core_axis_name="c", subcore_axis_name="s")
    kern = pl.kernel(
        functools.partial(_sc_gather_kernel,
                          rows_per_worker=rows_per_worker,
                          num_cores=info.num_cores),
        out_type=jax.ShapeDtypeStruct((K, BATCH, SEM_DIM), jnp.float32),
        mesh=mesh,
        scratch_types=[
            pltpu.VMEM((K, _NBR), jnp.int32),
            pltpu.VMEM((_NCH,), jnp.int32),
            pltpu.VMEM((_NCH, 128), jnp.float32),
            pltpu.SemaphoreType.DMA,
        ],
        compiler_params=pltpu.CompilerParams(use_tc_tiling_on_sc=True,
                                             needs_layout_passes=False),
    )
    return kern(table840, idx_t)


def kernel(avg_features, W, b, embed_table):
    tags_t, idx_t = _tc_call(avg_features, W, b)
    table840 = embed_table.reshape(NUM_CLASSES * 4, SEM_DIM // 4)
    semantic = _sc_gather(table840, idx_t)
    return tags_t.T, semantic.transpose(1, 0, 2)
